# baseline (speedup 1.0000x reference)
import functools

import jax
import jax.numpy as jnp
from jax import lax
from jax.experimental import pallas as pl
from jax.experimental.pallas import tpu as pltpu
from jax.experimental.pallas import tpu_sc as plsc

B = 16384
R = 64
NC = 2
NS = 16
NW = NC * NS
BPW = B // NW
CH = 128
NCH = BPW // CH
GROUPS = BPW // 16


def _cpd_sc_body(idx_hbm, e0_hbm, e1_hbm, e2_hbm, out_hbm,
                 idx_v, r0, r1, r2, out_v, sem):
    wid = lax.axis_index("s") * NC + lax.axis_index("c")

    pltpu.sync_copy(idx_hbm.at[wid], idx_v)

    copies = []
    for m, (tab, dst) in enumerate(((e0_hbm, r0), (e1_hbm, r1), (e2_hbm, r2))):
        for j in range(NCH):
            copies.append(
                pltpu.async_copy(tab.at[idx_v.at[m, j]],
                                 dst.at[pl.ds(j * CH, CH)], sem))
    for cp in copies:
        cp.wait()

    def group(g, carry):
        b0 = g * 16
        for rr in range(16):
            row = b0 + rr
            acc = None
            for c in range(4):
                a = r0[row, pl.ds(c * 16, 16)]
                bb = r1[row, pl.ds(c * 16, 16)]
                d = r2[row, pl.ds(c * 16, 16)]
                p = a * bb * d
                acc = p if acc is None else acc + p
            out_v[pl.ds(row * 16, 16)] = acc
        return carry

    lax.fori_loop(0, GROUPS, group, 0)

    pltpu.sync_copy(out_v, out_hbm.at[pl.ds(wid * BPW * 16, BPW * 16)])


def _reduce_tc_body(x_ref, o_ref):
    o_ref[:] = jnp.sum(x_ref[:], axis=1)


@jax.jit
def kernel(idxs, E0, E1, E2):
    idx_t = jnp.transpose(idxs.astype(jnp.int32), (1, 0))
    idx_t = idx_t.reshape(3, NW, NCH, CH).transpose(1, 0, 2, 3)

    mesh = plsc.VectorSubcoreMesh(core_axis_name="c", subcore_axis_name="s")
    sc_fn = pl.kernel(
        _cpd_sc_body,
        mesh=mesh,
        out_type=jax.ShapeDtypeStruct((B * 16,), jnp.float32),
        scratch_types=[
            pltpu.VMEM((3, NCH, CH), jnp.int32),
            pltpu.VMEM((BPW, R), jnp.float32),
            pltpu.VMEM((BPW, R), jnp.float32),
            pltpu.VMEM((BPW, R), jnp.float32),
            pltpu.VMEM((BPW * 16,), jnp.float32),
            pltpu.SemaphoreType.DMA,
        ],
        compiler_params=pltpu.CompilerParams(use_tc_tiling_on_sc=False),
    )
    partials = sc_fn(idx_t, E0, E1, E2).reshape(B, 16)

    rows_per_block = 2048
    out = pl.pallas_call(
        _reduce_tc_body,
        grid=(B // rows_per_block,),
        in_specs=[pl.BlockSpec((rows_per_block, 16), lambda i: (i, 0))],
        out_specs=pl.BlockSpec((rows_per_block,), lambda i: (i,)),
        out_shape=jax.ShapeDtypeStruct((B,), jnp.float32),
    )(partials)
    return out
